# baseline (speedup 1.0000x reference)
import functools

import jax
import jax.numpy as jnp
from jax import lax
from jax.experimental import pallas as pl
from jax.experimental.pallas import tpu as pltpu
from jax.experimental.pallas import tpu_sc as plsc

B = 16384
D = 128
L = 16
NC = 2
NS = 16
NW = NC * NS
BW = B // NW
C = 128
NCH = BW // C
NSLOT = 3
DC = D // L


def _sc_body(uid_hbm, iid_hbm, ut_hbm, it_hbm, w_hbm, b_hbm, out_hbm,
             uidx_v, iidx_v, ubuf, vbuf, pbuf, obuf, wbuf, bbuf, sem, sem_w):
    wid = lax.axis_index("s") * NC + lax.axis_index("c")

    cw = pltpu.async_copy(w_hbm, wbuf, sem_w)
    cb = pltpu.async_copy(b_hbm, bbuf, sem_w)
    cu = pltpu.async_copy(uid_hbm.at[wid], uidx_v, sem_w)
    ci = pltpu.async_copy(iid_hbm.at[wid], iidx_v, sem_w)
    cu.wait()
    ci.wait()

    def issue(k, s):
        pltpu.async_copy(ut_hbm.at[uidx_v.at[pl.ds(k * C, C)]],
                         ubuf.at[s], sem.at[0, s])
        pltpu.async_copy(it_hbm.at[iidx_v.at[pl.ds(k * C, C)]],
                         vbuf.at[s], sem.at[1, s])

    def wait(k, s):
        pltpu.make_async_copy(ut_hbm.at[uidx_v.at[pl.ds(k * C, C)]],
                              ubuf.at[s], sem.at[0, s]).wait()
        pltpu.make_async_copy(it_hbm.at[iidx_v.at[pl.ds(k * C, C)]],
                              vbuf.at[s], sem.at[1, s]).wait()

    issue(0, 0)
    issue(1, 1)
    cw.wait()
    cb.wait()

    lane = jnp.arange(L, dtype=jnp.int32)
    wsl = [wbuf[pl.ds(c * L, L)] for c in range(DC)]
    bias = bbuf[...]
    zero = jnp.zeros((L,), jnp.float32)

    def chunk_body(k, carry):
        s = lax.rem(k, NSLOT)

        @pl.when(k + 2 < NCH)
        def _():
            issue(k + 2, lax.rem(k + 2, NSLOT))

        wait(k, s)

        @plsc.parallel_loop(0, C, unroll=4)
        def row_body(r):
            t = [ubuf[s, r, pl.ds(c * L, L)] * vbuf[s, r, pl.ds(c * L, L)]
                 * wsl[c] for c in range(DC)]
            acc = (((t[0] + t[1]) + (t[2] + t[3]))
                   + ((t[4] + t[5]) + (t[6] + t[7])))
            plsc.store_scatter(pbuf, [lane, zero.astype(jnp.int32) + r], acc)

        @plsc.parallel_loop(0, DC, unroll=2)
        def grp_body(g):
            a = [pbuf[j, pl.ds(g * L, L)] for j in range(L)]
            for step in (8, 4, 2, 1):
                a = [a[j] + a[j + step] for j in range(step)]
            x = a[0] + bias
            obuf[pl.ds(k * C + g * L, L)] = 1.0 / (1.0 + jnp.exp(-x))

        return carry

    lax.fori_loop(0, NCH, chunk_body, 0)

    pltpu.sync_copy(obuf, out_hbm.at[pl.ds(wid * BW, BW)])


@functools.partial(
    pl.kernel,
    out_type=jax.ShapeDtypeStruct((B,), jnp.float32),
    mesh=plsc.VectorSubcoreMesh(core_axis_name="c", subcore_axis_name="s"),
    compiler_params=pltpu.CompilerParams(needs_layout_passes=False),
    scratch_types=[
        pltpu.VMEM((BW,), jnp.int32),
        pltpu.VMEM((BW,), jnp.int32),
        pltpu.VMEM((NSLOT, C, D), jnp.float32),
        pltpu.VMEM((NSLOT, C, D), jnp.float32),
        pltpu.VMEM((L, C), jnp.float32),
        pltpu.VMEM((BW,), jnp.float32),
        pltpu.VMEM((D,), jnp.float32),
        pltpu.VMEM((L,), jnp.float32),
        pltpu.SemaphoreType.DMA((2, NSLOT)),
        pltpu.SemaphoreType.DMA,
    ],
)
def _gmf_sc(uid, iid, ut, it, w, b, out, *scratch):
    _sc_body(uid, iid, ut, it, w, b, out, *scratch)


def kernel(user_ids, item_ids, user_table, item_table, W, b):
    return _gmf_sc(user_ids.astype(jnp.int32).reshape(NW, BW),
                   item_ids.astype(jnp.int32).reshape(NW, BW),
                   user_table, item_table, W.reshape(D),
                   jnp.broadcast_to(b.astype(jnp.float32), (L,)))


# speedup vs baseline: 1.0584x; 1.0584x over previous
import functools

import jax
import jax.numpy as jnp
from jax import lax
from jax.experimental import pallas as pl
from jax.experimental.pallas import tpu as pltpu
from jax.experimental.pallas import tpu_sc as plsc

B = 16384
D = 128
L = 16
NC = 2
NS = 16
NW = NC * NS
BW = B // NW
C = 128
NCH = BW // C
NSLOT = 2
DC = D // L


def _sc_body(uid_hbm, iid_hbm, ut_hbm, it_hbm, w_hbm, b_hbm, out_hbm,
             uidx_v, iidx_v, ubuf, vbuf, pbuf, obuf, wbuf, bbuf, sem, sem_w):
    wid = lax.axis_index("s") * NC + lax.axis_index("c")
    base = wid * BW

    cw = pltpu.async_copy(w_hbm, wbuf, sem_w)
    cb = pltpu.async_copy(b_hbm, bbuf.at[pl.ds(0, 1)], sem_w)
    cu = pltpu.async_copy(uid_hbm.at[pl.ds(base, BW)], uidx_v, sem_w)
    ci = pltpu.async_copy(iid_hbm.at[pl.ds(base, BW)], iidx_v, sem_w)
    cu.wait()
    ci.wait()

    def issue(k, s):
        pltpu.async_copy(ut_hbm.at[uidx_v.at[pl.ds(k * C, C)]],
                         ubuf.at[s], sem.at[0, s])
        pltpu.async_copy(it_hbm.at[iidx_v.at[pl.ds(k * C, C)]],
                         vbuf.at[s], sem.at[1, s])

    def wait(k, s):
        pltpu.make_async_copy(ut_hbm.at[uidx_v.at[pl.ds(k * C, C)]],
                              ubuf.at[s], sem.at[0, s]).wait()
        pltpu.make_async_copy(it_hbm.at[iidx_v.at[pl.ds(k * C, C)]],
                              vbuf.at[s], sem.at[1, s]).wait()

    issue(0, 0)
    cw.wait()
    cb.wait()

    lane = jnp.arange(L, dtype=jnp.int32)
    izero = jnp.zeros((L,), jnp.int32)
    wsl = [wbuf[0, pl.ds(c * L, L)] for c in range(DC)]
    bias = plsc.load_gather(bbuf, [izero])
    zero = jnp.zeros((L,), jnp.float32)

    def chunk_body(k, carry):
        s = lax.rem(k, NSLOT)

        @pl.when(k + 1 < NCH)
        def _():
            issue(k + 1, lax.rem(k + 1, NSLOT))

        wait(k, s)

        @plsc.parallel_loop(0, C, unroll=4)
        def row_body(r):
            t = [ubuf[s, r, pl.ds(c * L, L)] * vbuf[s, r, pl.ds(c * L, L)]
                 * wsl[c] for c in range(DC)]
            acc = (((t[0] + t[1]) + (t[2] + t[3]))
                   + ((t[4] + t[5]) + (t[6] + t[7])))
            plsc.store_scatter(pbuf, [lane, izero + r], acc)

        @plsc.parallel_loop(0, DC, unroll=2)
        def grp_body(g):
            a = [pbuf[j, pl.ds(g * L, L)] for j in range(L)]
            for step in (8, 4, 2, 1):
                a = [a[j] + a[j + step] for j in range(step)]
            x = a[0] + bias
            obuf[pl.ds(k * C + g * L, L)] = 1.0 / (1.0 + jnp.exp(-x))

        return carry

    lax.fori_loop(0, NCH, chunk_body, 0)

    pltpu.sync_copy(obuf, out_hbm.at[pl.ds(base, BW)])


@functools.partial(
    pl.kernel,
    out_type=jax.ShapeDtypeStruct((B,), jnp.float32),
    mesh=plsc.VectorSubcoreMesh(core_axis_name="c", subcore_axis_name="s"),
    compiler_params=pltpu.CompilerParams(needs_layout_passes=False),
    scratch_types=[
        pltpu.VMEM((BW,), jnp.int32),
        pltpu.VMEM((BW,), jnp.int32),
        pltpu.VMEM((NSLOT, C, D), jnp.float32),
        pltpu.VMEM((NSLOT, C, D), jnp.float32),
        pltpu.VMEM((L, C), jnp.float32),
        pltpu.VMEM((BW,), jnp.float32),
        pltpu.VMEM((1, D), jnp.float32),
        pltpu.VMEM((L,), jnp.float32),
        pltpu.SemaphoreType.DMA((2, NSLOT)),
        pltpu.SemaphoreType.DMA,
    ],
)
def _gmf_sc(uid, iid, ut, it, w, b, out, *scratch):
    _sc_body(uid, iid, ut, it, w, b, out, *scratch)


def kernel(user_ids, item_ids, user_table, item_table, W, b):
    return _gmf_sc(user_ids.astype(jnp.int32), item_ids.astype(jnp.int32),
                   user_table, item_table, W, b)


# speedup vs baseline: 1.1082x; 1.0470x over previous
import functools

import jax
import jax.numpy as jnp
from jax import lax
from jax.experimental import pallas as pl
from jax.experimental.pallas import tpu as pltpu
from jax.experimental.pallas import tpu_sc as plsc

B = 16384
D = 128
L = 16
NC = 2
NS = 16
NW = NC * NS
BW = B // NW
C = 128
NCH = BW // C
NSLOT = 2
DC = D // L


def _sc_body(uid_hbm, iid_hbm, ut_hbm, it_hbm, w_hbm, b_hbm, out_hbm,
             uidx_v, iidx_v, ubuf, vbuf, pbuf, obuf, wbuf, bbuf, sem, sem_w):
    wid = lax.axis_index("s") * NC + lax.axis_index("c")
    base = wid * BW

    cw = pltpu.async_copy(w_hbm, wbuf, sem_w.at[0])
    cb = pltpu.async_copy(b_hbm, bbuf.at[pl.ds(0, 1)], sem_w.at[1])
    cu = pltpu.async_copy(uid_hbm.at[pl.ds(base, BW)], uidx_v, sem_w.at[2])
    ci = pltpu.async_copy(iid_hbm.at[pl.ds(base, BW)], iidx_v, sem_w.at[3])
    cu.wait()
    ci.wait()

    def issue(k, s):
        pltpu.async_copy(ut_hbm.at[uidx_v.at[pl.ds(k * C, C)]],
                         ubuf.at[s], sem.at[0, s])
        pltpu.async_copy(it_hbm.at[iidx_v.at[pl.ds(k * C, C)]],
                         vbuf.at[s], sem.at[1, s])

    def wait(k, s):
        pltpu.make_async_copy(ut_hbm.at[uidx_v.at[pl.ds(k * C, C)]],
                              ubuf.at[s], sem.at[0, s]).wait()
        pltpu.make_async_copy(it_hbm.at[iidx_v.at[pl.ds(k * C, C)]],
                              vbuf.at[s], sem.at[1, s]).wait()

    issue(0, 0)
    cw.wait()
    cb.wait()

    lane = jnp.arange(L, dtype=jnp.int32)
    izero = jnp.zeros((L,), jnp.int32)
    wsl = [wbuf[0, pl.ds(c * L, L)] for c in range(DC)]
    bias = plsc.load_gather(bbuf, [izero])
    zero = jnp.zeros((L,), jnp.float32)

    def chunk_body(k, carry):
        s = lax.rem(k, NSLOT)

        @pl.when(k + 1 < NCH)
        def _():
            issue(k + 1, lax.rem(k + 1, NSLOT))

        wait(k, s)

        @plsc.parallel_loop(0, C, unroll=4)
        def row_body(r):
            t = [ubuf[s, r, pl.ds(c * L, L)] * vbuf[s, r, pl.ds(c * L, L)]
                 * wsl[c] for c in range(DC)]
            acc = (((t[0] + t[1]) + (t[2] + t[3]))
                   + ((t[4] + t[5]) + (t[6] + t[7])))
            plsc.store_scatter(pbuf, [lane, izero + r], acc)

        @plsc.parallel_loop(0, DC, unroll=2)
        def grp_body(g):
            a = [pbuf[j, pl.ds(g * L, L)] for j in range(L)]
            for step in (8, 4, 2, 1):
                a = [a[j] + a[j + step] for j in range(step)]
            x = a[0] + bias
            obuf[pl.ds(k * C + g * L, L)] = 1.0 / (1.0 + jnp.exp(-x))

        return carry

    lax.fori_loop(0, NCH, chunk_body, 0)

    pltpu.sync_copy(obuf, out_hbm.at[pl.ds(base, BW)])


@functools.partial(
    pl.kernel,
    out_type=jax.ShapeDtypeStruct((B,), jnp.float32),
    mesh=plsc.VectorSubcoreMesh(core_axis_name="c", subcore_axis_name="s"),
    compiler_params=pltpu.CompilerParams(needs_layout_passes=False),
    scratch_types=[
        pltpu.VMEM((BW,), jnp.int32),
        pltpu.VMEM((BW,), jnp.int32),
        pltpu.VMEM((NSLOT, C, D), jnp.float32),
        pltpu.VMEM((NSLOT, C, D), jnp.float32),
        pltpu.VMEM((L, C), jnp.float32),
        pltpu.VMEM((BW,), jnp.float32),
        pltpu.VMEM((1, D), jnp.float32),
        pltpu.VMEM((L,), jnp.float32),
        pltpu.SemaphoreType.DMA((2, NSLOT)),
        pltpu.SemaphoreType.DMA((4,)),
    ],
)
def _gmf_sc(uid, iid, ut, it, w, b, out, *scratch):
    _sc_body(uid, iid, ut, it, w, b, out, *scratch)


def kernel(user_ids, item_ids, user_table, item_table, W, b):
    return _gmf_sc(user_ids.astype(jnp.int32), item_ids.astype(jnp.int32),
                   user_table, item_table, W, b)


# speedup vs baseline: 1.1175x; 1.0084x over previous
import functools

import jax
import jax.numpy as jnp
from jax import lax
from jax.experimental import pallas as pl
from jax.experimental.pallas import tpu as pltpu
from jax.experimental.pallas import tpu_sc as plsc

B = 16384
D = 128
L = 16
NC = 2
NS = 16
NW = NC * NS
BW = B // NW
C = 128
NCH = BW // C
NSLOT = 2
DC = D // L


def _sc_body(uid_hbm, iid_hbm, ut_hbm, it_hbm, w_hbm, b_hbm, out_hbm,
             uidx_v, iidx_v, ubuf, vbuf, pbuf, obuf, wbuf, bbuf, sem, sem_w):
    wid = lax.axis_index("s") * NC + lax.axis_index("c")
    base = wid * BW

    cw = pltpu.async_copy(w_hbm, wbuf, sem_w.at[0])
    cb = pltpu.async_copy(b_hbm, bbuf, sem_w.at[1])
    cu = pltpu.async_copy(uid_hbm.at[pl.ds(base, BW)], uidx_v, sem_w.at[2])
    ci = pltpu.async_copy(iid_hbm.at[pl.ds(base, BW)], iidx_v, sem_w.at[3])
    cu.wait()
    ci.wait()

    def issue(k, s):
        pltpu.async_copy(ut_hbm.at[uidx_v.at[pl.ds(k * C, C)]],
                         ubuf.at[s], sem.at[0, s])
        pltpu.async_copy(it_hbm.at[iidx_v.at[pl.ds(k * C, C)]],
                         vbuf.at[s], sem.at[1, s])

    def wait(k, s):
        pltpu.make_async_copy(ut_hbm.at[uidx_v.at[pl.ds(k * C, C)]],
                              ubuf.at[s], sem.at[0, s]).wait()
        pltpu.make_async_copy(it_hbm.at[iidx_v.at[pl.ds(k * C, C)]],
                              vbuf.at[s], sem.at[1, s]).wait()

    issue(0, 0)
    cw.wait()
    cb.wait()

    lane = jnp.arange(L, dtype=jnp.int32)
    izero = jnp.zeros((L,), jnp.int32)
    wsl = [wbuf[0, pl.ds(c * L, L)] for c in range(DC)]
    bias = bbuf[...]
    zero = jnp.zeros((L,), jnp.float32)

    def chunk_body(k, carry):
        s = lax.rem(k, NSLOT)

        @pl.when(k + 1 < NCH)
        def _():
            issue(k + 1, lax.rem(k + 1, NSLOT))

        wait(k, s)

        @plsc.parallel_loop(0, C, unroll=4)
        def row_body(r):
            t = [ubuf[s, r, pl.ds(c * L, L)] * vbuf[s, r, pl.ds(c * L, L)]
                 * wsl[c] for c in range(DC)]
            acc = (((t[0] + t[1]) + (t[2] + t[3]))
                   + ((t[4] + t[5]) + (t[6] + t[7])))
            plsc.store_scatter(pbuf, [lane, izero + r], acc)

        @plsc.parallel_loop(0, DC, unroll=2)
        def grp_body(g):
            a = [pbuf[j, pl.ds(g * L, L)] for j in range(L)]
            for step in (8, 4, 2, 1):
                a = [a[j] + a[j + step] for j in range(step)]
            x = a[0] + bias
            obuf[pl.ds(k * C + g * L, L)] = 1.0 / (1.0 + jnp.exp(-x))

        return carry

    lax.fori_loop(0, NCH, chunk_body, 0)

    pltpu.sync_copy(obuf, out_hbm.at[pl.ds(base, BW)])


@functools.partial(
    pl.kernel,
    out_type=jax.ShapeDtypeStruct((B,), jnp.float32),
    mesh=plsc.VectorSubcoreMesh(core_axis_name="c", subcore_axis_name="s"),
    compiler_params=pltpu.CompilerParams(needs_layout_passes=False),
    scratch_types=[
        pltpu.VMEM((BW,), jnp.int32),
        pltpu.VMEM((BW,), jnp.int32),
        pltpu.VMEM((NSLOT, C, D), jnp.float32),
        pltpu.VMEM((NSLOT, C, D), jnp.float32),
        pltpu.VMEM((L, C), jnp.float32),
        pltpu.VMEM((BW,), jnp.float32),
        pltpu.VMEM((1, D), jnp.float32),
        pltpu.VMEM((L,), jnp.float32),
        pltpu.SemaphoreType.DMA((2, NSLOT)),
        pltpu.SemaphoreType.DMA((4,)),
    ],
)
def _gmf_sc(uid, iid, ut, it, w, b, out, *scratch):
    _sc_body(uid, iid, ut, it, w, b, out, *scratch)


def kernel(user_ids, item_ids, user_table, item_table, W, b):
    return _gmf_sc(user_ids.astype(jnp.int32), item_ids.astype(jnp.int32),
                   user_table, item_table, W,
                   jnp.broadcast_to(b.astype(jnp.float32), (L,)))


# speedup vs baseline: 1.1251x; 1.0068x over previous
import functools

import jax
import jax.numpy as jnp
from jax import lax
from jax.experimental import pallas as pl
from jax.experimental.pallas import tpu as pltpu
from jax.experimental.pallas import tpu_sc as plsc

B = 16384
D = 128
L = 16
NC = 2
NS = 16
NW = NC * NS
BW = B // NW
C = 64
NCH = BW // C
NSLOT = 4
LOOK = NSLOT - 1
NG = C // L
DC = D // L


def _sc_body(uid_hbm, iid_hbm, ut_hbm, it_hbm, w_hbm, b_hbm, out_hbm,
             uidx_v, iidx_v, ubuf, vbuf, pbuf, obuf, wbuf, bbuf, sem, sem_w):
    wid = lax.axis_index("s") * NC + lax.axis_index("c")
    base = wid * BW

    cw = pltpu.async_copy(w_hbm, wbuf, sem_w.at[0])
    cb = pltpu.async_copy(b_hbm, bbuf, sem_w.at[1])
    cu = pltpu.async_copy(uid_hbm.at[pl.ds(base, BW)], uidx_v, sem_w.at[2])
    ci = pltpu.async_copy(iid_hbm.at[pl.ds(base, BW)], iidx_v, sem_w.at[3])
    cu.wait()
    ci.wait()

    def issue(k, s):
        pltpu.async_copy(ut_hbm.at[uidx_v.at[pl.ds(k * C, C)]],
                         ubuf.at[s], sem.at[0, s])
        pltpu.async_copy(it_hbm.at[iidx_v.at[pl.ds(k * C, C)]],
                         vbuf.at[s], sem.at[1, s])

    def wait(k, s):
        pltpu.make_async_copy(ut_hbm.at[uidx_v.at[pl.ds(k * C, C)]],
                              ubuf.at[s], sem.at[0, s]).wait()
        pltpu.make_async_copy(it_hbm.at[iidx_v.at[pl.ds(k * C, C)]],
                              vbuf.at[s], sem.at[1, s]).wait()

    for k0 in range(LOOK):
        issue(k0, k0)
    cw.wait()
    cb.wait()

    lane = jnp.arange(L, dtype=jnp.int32)
    izero = jnp.zeros((L,), jnp.int32)
    wsl = [wbuf[0, pl.ds(c * L, L)] for c in range(DC)]
    bias = bbuf[...]
    zero = jnp.zeros((L,), jnp.float32)

    def chunk_body(k, carry):
        s = lax.rem(k, NSLOT)

        @pl.when(k + LOOK < NCH)
        def _():
            issue(k + LOOK, lax.rem(k + LOOK, NSLOT))

        wait(k, s)

        @plsc.parallel_loop(0, C, unroll=4)
        def row_body(r):
            t = [ubuf[s, r, pl.ds(c * L, L)] * vbuf[s, r, pl.ds(c * L, L)]
                 * wsl[c] for c in range(DC)]
            acc = (((t[0] + t[1]) + (t[2] + t[3]))
                   + ((t[4] + t[5]) + (t[6] + t[7])))
            plsc.store_scatter(pbuf, [lane, izero + r], acc)

        @plsc.parallel_loop(0, NG, unroll=2)
        def grp_body(g):
            a = [pbuf[j, pl.ds(g * L, L)] for j in range(L)]
            for step in (8, 4, 2, 1):
                a = [a[j] + a[j + step] for j in range(step)]
            x = a[0] + bias
            obuf[pl.ds(k * C + g * L, L)] = 1.0 / (1.0 + jnp.exp(-x))

        return carry

    lax.fori_loop(0, NCH, chunk_body, 0)

    pltpu.sync_copy(obuf, out_hbm.at[pl.ds(base, BW)])


@functools.partial(
    pl.kernel,
    out_type=jax.ShapeDtypeStruct((B,), jnp.float32),
    mesh=plsc.VectorSubcoreMesh(core_axis_name="c", subcore_axis_name="s"),
    compiler_params=pltpu.CompilerParams(needs_layout_passes=False),
    scratch_types=[
        pltpu.VMEM((BW,), jnp.int32),
        pltpu.VMEM((BW,), jnp.int32),
        pltpu.VMEM((NSLOT, C, D), jnp.float32),
        pltpu.VMEM((NSLOT, C, D), jnp.float32),
        pltpu.VMEM((L, C), jnp.float32),
        pltpu.VMEM((BW,), jnp.float32),
        pltpu.VMEM((1, D), jnp.float32),
        pltpu.VMEM((L,), jnp.float32),
        pltpu.SemaphoreType.DMA((2, NSLOT)),
        pltpu.SemaphoreType.DMA((4,)),
    ],
)
def _gmf_sc(uid, iid, ut, it, w, b, out, *scratch):
    _sc_body(uid, iid, ut, it, w, b, out, *scratch)


def kernel(user_ids, item_ids, user_table, item_table, W, b):
    return _gmf_sc(user_ids.astype(jnp.int32), item_ids.astype(jnp.int32),
                   user_table, item_table, W,
                   jnp.broadcast_to(b.astype(jnp.float32), (L,)))


# speedup vs baseline: 1.1618x; 1.0326x over previous
import functools

import jax
import jax.numpy as jnp
from jax import lax
from jax.experimental import pallas as pl
from jax.experimental.pallas import tpu as pltpu
from jax.experimental.pallas import tpu_sc as plsc

B = 16384
D = 128
L = 16
NC = 2
NS = 16
NW = NC * NS
BW = B // NW
C = 32
NCH = BW // C
NSLOT = 8
LOOK = NSLOT - 1
NG = C // L
DC = D // L


def _sc_body(uid_hbm, iid_hbm, ut_hbm, it_hbm, w_hbm, b_hbm, out_hbm,
             uidx_v, iidx_v, ubuf, vbuf, pbuf, obuf, wbuf, bbuf, sem, sem_w):
    wid = lax.axis_index("s") * NC + lax.axis_index("c")
    base = wid * BW

    cw = pltpu.async_copy(w_hbm, wbuf, sem_w.at[0])
    cb = pltpu.async_copy(b_hbm, bbuf, sem_w.at[1])
    c0u = pltpu.async_copy(uid_hbm.at[pl.ds(base, C)],
                           uidx_v.at[pl.ds(0, C)], sem_w.at[2])
    c0i = pltpu.async_copy(iid_hbm.at[pl.ds(base, C)],
                           iidx_v.at[pl.ds(0, C)], sem_w.at[3])
    c0u.wait()
    c0i.wait()
    cu = pltpu.async_copy(uid_hbm.at[pl.ds(base + C, BW - C)],
                          uidx_v.at[pl.ds(C, BW - C)], sem_w.at[2])
    ci = pltpu.async_copy(iid_hbm.at[pl.ds(base + C, BW - C)],
                          iidx_v.at[pl.ds(C, BW - C)], sem_w.at[3])

    def issue(k, s):
        pltpu.async_copy(ut_hbm.at[uidx_v.at[pl.ds(k * C, C)]],
                         ubuf.at[s], sem.at[0, s])
        pltpu.async_copy(it_hbm.at[iidx_v.at[pl.ds(k * C, C)]],
                         vbuf.at[s], sem.at[1, s])

    def wait(k, s):
        pltpu.make_async_copy(ut_hbm.at[uidx_v.at[pl.ds(k * C, C)]],
                              ubuf.at[s], sem.at[0, s]).wait()
        pltpu.make_async_copy(it_hbm.at[iidx_v.at[pl.ds(k * C, C)]],
                              vbuf.at[s], sem.at[1, s]).wait()

    issue(0, 0)
    cu.wait()
    ci.wait()
    for k0 in range(1, LOOK):
        issue(k0, k0)
    cw.wait()
    cb.wait()

    lane = jnp.arange(L, dtype=jnp.int32)
    izero = jnp.zeros((L,), jnp.int32)
    wsl = [wbuf[0, pl.ds(c * L, L)] for c in range(DC)]
    bias = bbuf[...]
    zero = jnp.zeros((L,), jnp.float32)

    def chunk_body(k, carry):
        s = lax.rem(k, NSLOT)

        @pl.when(k + LOOK < NCH)
        def _():
            issue(k + LOOK, lax.rem(k + LOOK, NSLOT))

        wait(k, s)

        @plsc.parallel_loop(0, C, unroll=4)
        def row_body(r):
            t = [ubuf[s, r, pl.ds(c * L, L)] * vbuf[s, r, pl.ds(c * L, L)]
                 * wsl[c] for c in range(DC)]
            acc = (((t[0] + t[1]) + (t[2] + t[3]))
                   + ((t[4] + t[5]) + (t[6] + t[7])))
            plsc.store_scatter(pbuf, [lane, izero + r], acc)

        @plsc.parallel_loop(0, NG, unroll=2)
        def grp_body(g):
            a = [pbuf[j, pl.ds(g * L, L)] for j in range(L)]
            for step in (8, 4, 2, 1):
                a = [a[j] + a[j + step] for j in range(step)]
            x = a[0] + bias
            obuf[pl.ds(k * C + g * L, L)] = 1.0 / (1.0 + jnp.exp(-x))

        return carry

    lax.fori_loop(0, NCH, chunk_body, 0)

    pltpu.sync_copy(obuf, out_hbm.at[pl.ds(base, BW)])


@functools.partial(
    pl.kernel,
    out_type=jax.ShapeDtypeStruct((B,), jnp.float32),
    mesh=plsc.VectorSubcoreMesh(core_axis_name="c", subcore_axis_name="s"),
    compiler_params=pltpu.CompilerParams(needs_layout_passes=False),
    scratch_types=[
        pltpu.VMEM((BW,), jnp.int32),
        pltpu.VMEM((BW,), jnp.int32),
        pltpu.VMEM((NSLOT, C, D), jnp.float32),
        pltpu.VMEM((NSLOT, C, D), jnp.float32),
        pltpu.VMEM((L, C), jnp.float32),
        pltpu.VMEM((BW,), jnp.float32),
        pltpu.VMEM((1, D), jnp.float32),
        pltpu.VMEM((L,), jnp.float32),
        pltpu.SemaphoreType.DMA((2, NSLOT)),
        pltpu.SemaphoreType.DMA((4,)),
    ],
)
def _gmf_sc(uid, iid, ut, it, w, b, out, *scratch):
    _sc_body(uid, iid, ut, it, w, b, out, *scratch)


def kernel(user_ids, item_ids, user_table, item_table, W, b):
    return _gmf_sc(user_ids.astype(jnp.int32), item_ids.astype(jnp.int32),
                   user_table, item_table, W,
                   jnp.broadcast_to(b.astype(jnp.float32), (L,)))
